# Initial kernel scaffold; baseline (speedup 1.0000x reference)
import jax
import jax.numpy as jnp
from jax.experimental import pallas as pl


def kernel(node_embeddings, batch, gate_w, gate_b, w1, b1, w2, b2):
    raise NotImplementedError("write your pallas kernel here")


# speedup vs baseline: 2.1674x; 2.1674x over previous
import functools

import jax
import jax.numpy as jnp
from jax import lax
from jax.experimental import pallas as pl
from jax.experimental.pallas import tpu as pltpu
from jax.experimental.pallas import tpu_sc as plsc

N, D, C, G = 50000, 256, 2, 64
NC, NS = 2, 16
NW = NC * NS
CHUNK = 1568
R = 112
DC = D // 16


def _sc_body(x_hbm, b_hbm, gw_hbm, gb_hbm, parts_hbm, att_hbm,
             xbuf, bbuf, attb, gwv, gbv, acc):
    c = lax.axis_index("c")
    s = lax.axis_index("s")
    w = s * NC + c
    start = w * CHUNK
    end = jnp.minimum(start + CHUNK, N)

    pltpu.sync_copy(gw_hbm, gwv)
    pltpu.sync_copy(gb_hbm, gbv)

    def _zero(i, carry):
        acc[pl.ds(i * 16, 16)] = jnp.zeros((16,), jnp.float32)
        return carry
    lax.fori_loop(0, (G * D) // 16, _zero, 0)

    gw_regs = [gwv[pl.ds(k * 16, 16)] for k in range(DC)]
    gb_reg = gbv[...]
    iota = lax.iota(jnp.int32, 16)
    xor_perms = [(iota ^ bit) for bit in (1, 2, 4, 8)]

    def _allsum(v):
        for perm in xor_perms:
            v = v + v.at[perm].get(mode="promise_in_bounds")
        return v

    def _block(bk, carry):
        p = start + bk * R
        base = pl.multiple_of(jnp.minimum(p, end - R), 16)
        off = p - base
        pltpu.sync_copy(x_hbm.at[pl.ds(base * D, R * D)], xbuf)
        pltpu.sync_copy(b_hbm.at[pl.ds(base, R)], bbuf)

        def _group(gi, carry):
            bvec = bbuf[pl.ds(gi * 16, 16)]
            att_g = jnp.zeros((16,), jnp.float32)
            for j in range(16):
                row = gi * 16 + j
                rbase = row * D
                xs = [xbuf[pl.ds(rbase + k * 16, 16)] for k in range(DC)]
                dot = xs[0] * gw_regs[0]
                for k in range(1, DC):
                    dot = dot + xs[k] * gw_regs[k]
                z = _allsum(dot) + gb_reg
                att_v = 1.0 / (1.0 + jnp.exp(-z))
                att_g = jnp.where(iota == j, att_v, att_g)
                maskv = jnp.full((16,), row, jnp.int32) >= jnp.full(
                    (16,), off, jnp.int32)
                att_e = jnp.where(maskv, att_v, jnp.zeros((16,), jnp.float32))
                gs = bvec.at[jnp.full((16,), j, jnp.int32)].get(
                    mode="promise_in_bounds")
                gbase = gs * D + iota
                for k in range(DC):
                    plsc.addupdate_scatter(acc, [gbase + k * 16],
                                           att_e * xs[k])
            attb[pl.ds(base - start + gi * 16, 16)] = att_g
            return carry
        lax.fori_loop(0, R // 16, _group, 0)
        return carry

    lax.fori_loop(0, CHUNK // R, _block, 0)

    pltpu.sync_copy(acc, parts_hbm.at[pl.ds(w * G * D, G * D)])
    pltpu.sync_copy(attb, att_hbm.at[pl.ds(w * CHUNK, CHUNK)])


_sc_call = pl.kernel(
    _sc_body,
    out_type=(
        jax.ShapeDtypeStruct((NW * G * D,), jnp.float32),
        jax.ShapeDtypeStruct((NW * CHUNK,), jnp.float32),
    ),
    mesh=plsc.VectorSubcoreMesh(core_axis_name="c", subcore_axis_name="s"),
    compiler_params=pltpu.CompilerParams(needs_layout_passes=False),
    scratch_types=[
        pltpu.VMEM((R * D,), jnp.float32),
        pltpu.VMEM((R,), jnp.int32),
        pltpu.VMEM((CHUNK,), jnp.float32),
        pltpu.VMEM((D,), jnp.float32),
        pltpu.VMEM((16,), jnp.float32),
        pltpu.VMEM((G * D,), jnp.float32),
    ],
)


def _tc_body(parts_ref, w1_ref, b1_ref, w2_ref, b2_ref, out_ref):
    ge = jnp.sum(parts_ref[...], axis=0)
    h = jnp.maximum(
        jnp.dot(ge, w1_ref[...], preferred_element_type=jnp.float32,
                precision=lax.Precision.HIGHEST)
        + b1_ref[...], 0.0)
    out_ref[...] = (
        jnp.dot(h, w2_ref[...], preferred_element_type=jnp.float32,
                precision=lax.Precision.HIGHEST)
        + b2_ref[...])


_tc_call = pl.pallas_call(
    _tc_body,
    out_shape=jax.ShapeDtypeStruct((G, C), jnp.float32),
)


def kernel(node_embeddings, batch, gate_w, gate_b, w1, b1, w2, b2):
    x_flat = node_embeddings.reshape(N * D)
    b_i32 = batch.astype(jnp.int32)
    gw_flat = gate_w.reshape(D)
    gb16 = jnp.broadcast_to(gate_b.reshape(1), (16,))
    parts, att_full = _sc_call(x_flat, b_i32, gw_flat, gb16)
    att = att_full[:N].reshape(N, 1)
    logits = _tc_call(parts.reshape(NW, G, D), w1, b1.reshape(1, D),
                      w2, b2.reshape(1, C))
    return (logits, att)


# speedup vs baseline: 2.4856x; 1.1469x over previous
import functools

import jax
import jax.numpy as jnp
from jax import lax
from jax.experimental import pallas as pl
from jax.experimental.pallas import tpu as pltpu
from jax.experimental.pallas import tpu_sc as plsc

N, D, C, G = 50000, 256, 2, 64
NC, NS = 2, 16
NW = NC * NS
CHUNK = 1568
R = 112
DC = D // 16


def _sc_body(x_hbm, b_hbm, gw_hbm, gb_hbm, parts_hbm, att_hbm,
             xbuf0, xbuf1, bbuf0, bbuf1, attb, gwv, gbv, acc,
             sem0, sem1):
    c = lax.axis_index("c")
    s = lax.axis_index("s")
    w = s * NC + c
    start = w * CHUNK
    end = jnp.minimum(start + CHUNK, N)

    pltpu.sync_copy(gw_hbm, gwv)
    pltpu.sync_copy(gb_hbm, gbv)

    def _zero(i, carry):
        acc[pl.ds(i * 16, 16)] = jnp.zeros((16,), jnp.float32)
        return carry
    lax.fori_loop(0, (G * D) // 16, _zero, 0)

    gw_regs = [gwv[pl.ds(k * 16, 16)] for k in range(DC)]
    gb_reg = gbv[...]
    iota = lax.iota(jnp.int32, 16)
    xor_perms = [(iota ^ bit) for bit in (1, 2, 4, 8)]

    def _allsum(v):
        for perm in xor_perms:
            v = v + v.at[perm].get(mode="promise_in_bounds")
        return v

    def _base_of(bk):
        p = start + bk * R
        base = pl.multiple_of(jnp.minimum(p, end - R), 16)
        return base, p - base

    def _start_fetch(bk, xbuf, bbuf, sem):
        base, _ = _base_of(bk)
        pltpu.async_copy(x_hbm.at[pl.ds(base * D, R * D)], xbuf, sem)
        pltpu.async_copy(b_hbm.at[pl.ds(base, R)], bbuf, sem)

    def _wait_fetch(xbuf, bbuf, sem):
        pltpu.make_async_copy(x_hbm.at[pl.ds(0, R * D)], xbuf, sem).wait()
        pltpu.make_async_copy(b_hbm.at[pl.ds(0, R)], bbuf, sem).wait()

    def _process(bk, xbuf, bbuf):
        base, off = _base_of(bk)

        def _group(gi, carry):
            bvec = bbuf[pl.ds(gi * 16, 16)]
            att_g = jnp.zeros((16,), jnp.float32)
            for j in range(16):
                row = gi * 16 + j
                rbase = row * D
                xs = [xbuf[pl.ds(rbase + k * 16, 16)] for k in range(DC)]
                dot = xs[0] * gw_regs[0]
                for k in range(1, DC):
                    dot = dot + xs[k] * gw_regs[k]
                z = _allsum(dot) + gb_reg
                att_v = 1.0 / (1.0 + jnp.exp(-z))
                att_g = jnp.where(iota == j, att_v, att_g)
                maskv = jnp.full((16,), row, jnp.int32) >= jnp.full(
                    (16,), off, jnp.int32)
                att_e = jnp.where(maskv, att_v, jnp.zeros((16,), jnp.float32))
                gs = bvec.at[jnp.full((16,), j, jnp.int32)].get(
                    mode="promise_in_bounds")
                gbase = gs * D + iota
                for k in range(DC):
                    plsc.addupdate_scatter(acc, [gbase + k * 16],
                                           att_e * xs[k])
            attb[pl.ds(base - start + gi * 16, 16)] = att_g
            return carry
        lax.fori_loop(0, R // 16, _group, 0)

    _start_fetch(0, xbuf0, bbuf0, sem0)

    def _pair(i, carry):
        k0 = 2 * i
        _wait_fetch(xbuf0, bbuf0, sem0)
        _start_fetch(k0 + 1, xbuf1, bbuf1, sem1)
        _process(k0, xbuf0, bbuf0)
        _wait_fetch(xbuf1, bbuf1, sem1)
        _start_fetch(k0 + 2, xbuf0, bbuf0, sem0)
        _process(k0 + 1, xbuf1, bbuf1)
        return carry

    lax.fori_loop(0, (CHUNK // R) // 2, _pair, 0)
    _wait_fetch(xbuf0, bbuf0, sem0)

    pltpu.sync_copy(acc, parts_hbm.at[pl.ds(w * G * D, G * D)])
    pltpu.sync_copy(attb, att_hbm.at[pl.ds(w * CHUNK, CHUNK)])


_sc_call = pl.kernel(
    _sc_body,
    out_type=(
        jax.ShapeDtypeStruct((NW * G * D,), jnp.float32),
        jax.ShapeDtypeStruct((NW * CHUNK,), jnp.float32),
    ),
    mesh=plsc.VectorSubcoreMesh(core_axis_name="c", subcore_axis_name="s"),
    compiler_params=pltpu.CompilerParams(needs_layout_passes=False),
    scratch_types=[
        pltpu.VMEM((R * D,), jnp.float32),
        pltpu.VMEM((R * D,), jnp.float32),
        pltpu.VMEM((R,), jnp.int32),
        pltpu.VMEM((R,), jnp.int32),
        pltpu.VMEM((CHUNK,), jnp.float32),
        pltpu.VMEM((D,), jnp.float32),
        pltpu.VMEM((16,), jnp.float32),
        pltpu.VMEM((G * D,), jnp.float32),
        pltpu.SemaphoreType.DMA,
        pltpu.SemaphoreType.DMA,
    ],
)


def _tc_body(parts_ref, att_in_ref, w1_ref, b1_ref, w2_ref, b2_ref,
             out_ref, att_out_ref):
    ge = jnp.sum(parts_ref[...], axis=0)
    h = jnp.maximum(
        jnp.dot(ge, w1_ref[...], preferred_element_type=jnp.float32)
        + b1_ref[...], 0.0)
    out_ref[...] = (
        jnp.dot(h, w2_ref[...], preferred_element_type=jnp.float32)
        + b2_ref[...])
    att_out_ref[...] = att_in_ref[0:N]


_tc_call = pl.pallas_call(
    _tc_body,
    out_shape=(
        jax.ShapeDtypeStruct((G, C), jnp.float32),
        jax.ShapeDtypeStruct((N,), jnp.float32),
    ),
)


def kernel(node_embeddings, batch, gate_w, gate_b, w1, b1, w2, b2):
    x_flat = node_embeddings.reshape(N * D)
    b_i32 = batch.astype(jnp.int32)
    gw_flat = gate_w.reshape(D)
    gb16 = jnp.broadcast_to(gate_b.reshape(1), (16,))
    parts, att_full = _sc_call(x_flat, b_i32, gw_flat, gb16)
    logits, att1d = _tc_call(parts.reshape(NW, G, D), att_full,
                             w1, b1.reshape(1, D), w2, b2.reshape(1, C))
    return (logits, att1d.reshape(N, 1))
